# baseline (speedup 1.0000x reference)
import functools

import jax
import jax.numpy as jnp
from jax import lax
from jax.experimental import pallas as pl
from jax.experimental.pallas import tpu as pltpu
from jax.experimental.pallas import tpu_sc as plsc

D_MODEL = 768
SEQ = 2048
BATCH = 4
NC = 2
NS = 16
L = 16
NW = NC * NS
POS_PER_W = SEQ // NW
CHUNK = 16
POS_CHUNKS = POS_PER_W // CHUNK
N_CHUNKS = BATCH * POS_CHUNKS
VECS_PER_ROW = D_MODEL // L
N_ROWS_BUF = 4
N_EMB_BUF = 3
N_POS_BUF = 2


def _sc_embed(idx_hbm, table_hbm, pos_hbm):
    mesh = plsc.VectorSubcoreMesh(core_axis_name="c", subcore_axis_name="s")

    @functools.partial(
        pl.kernel,
        out_type=(
            jax.ShapeDtypeStruct((BATCH * SEQ, D_MODEL), jnp.float32),
            jax.ShapeDtypeStruct((BATCH * SEQ, D_MODEL), jnp.float32),
        ),
        mesh=mesh,
        scratch_types=[
            pltpu.VMEM((BATCH, POS_PER_W), jnp.int32),
            [pltpu.VMEM((CHUNK, D_MODEL), jnp.float32)] * N_POS_BUF,
            [pltpu.VMEM((CHUNK, D_MODEL), jnp.float32)] * N_ROWS_BUF,
            [pltpu.VMEM((CHUNK, D_MODEL), jnp.float32)] * N_EMB_BUF,
            pltpu.SemaphoreType.DMA,
            pltpu.SemaphoreType.DMA,
            [pltpu.SemaphoreType.DMA] * N_ROWS_BUF,
            [pltpu.SemaphoreType.DMA] * N_ROWS_BUF,
            [pltpu.SemaphoreType.DMA] * N_EMB_BUF,
        ],
    )
    def k(idx_ref, table_ref, pos_ref, emb_out, wo_out,
          idx_v, pos_v, rows_v, emb_v, isem, psem, gsem, wsem, esem):
        wid = lax.axis_index("s") * NC + lax.axis_index("c")
        pos_base = wid * POS_PER_W

        def pos_load(c):
            return pltpu.async_copy(
                pos_ref.at[pl.ds(pos_base + c * CHUNK, CHUNK)],
                pos_v[c % N_POS_BUF], psem)

        def gather(i):
            c, b = divmod(i, BATCH)
            p = i % N_ROWS_BUF
            return pltpu.async_copy(
                table_ref.at[idx_v.at[b, pl.ds(c * CHUNK, CHUNK)]],
                rows_v[p], gsem[p])

        p_cp = {0: pos_load(0)}
        idx_cps = [
            pltpu.async_copy(
                idx_ref.at[b, pl.ds(pos_base, POS_PER_W)], idx_v.at[b], isem)
            for b in range(BATCH)
        ]
        for cp in idx_cps:
            cp.wait()

        g_cp = {0: gather(0), 1: gather(1)}
        w_cp = {}
        e_cp = {}
        for i in range(N_CHUNKS):
            p = i % N_ROWS_BUF
            q = i % N_EMB_BUF
            c, b = divmod(i, BATCH)
            flat = b * SEQ + pos_base + c * CHUNK

            if i + 2 < N_CHUNKS:
                if i - 2 >= 0:
                    w_cp.pop(i - 2).wait()
                g_cp[i + 2] = gather(i + 2)

            if b == 0 and c + 1 < POS_CHUNKS:
                p_cp[c + 1] = pos_load(c + 1)

            g_cp.pop(i).wait()
            rows_ref = rows_v[p]
            w_cp[i] = pltpu.async_copy(
                rows_ref, wo_out.at[pl.ds(flat, CHUNK)], wsem[p])

            if i - N_EMB_BUF >= 0:
                e_cp.pop(i - N_EMB_BUF).wait()
            if b == 0:
                p_cp.pop(c).wait()

            emb_ref = emb_v[q]
            pos_chunk = pos_v[c % N_POS_BUF]

            def body(r, carry, rows_ref=rows_ref, emb_ref=emb_ref,
                     pos_chunk=pos_chunk):
                for kk in range(VECS_PER_ROW):
                    sl = pl.ds(kk * L, L)
                    emb_ref[r, sl] = rows_ref[r, sl] + pos_chunk[r, sl]
                return carry

            lax.fori_loop(0, CHUNK, body, 0)

            e_cp[i] = pltpu.async_copy(
                emb_ref, emb_out.at[pl.ds(flat, CHUNK)], esem[q])

        for i in sorted(w_cp):
            w_cp[i].wait()
        for i in sorted(e_cp):
            e_cp[i].wait()

    return k(idx_hbm, table_hbm, pos_hbm)


def kernel(inputs, token_embeddings, position_embeddings):
    idx = inputs.astype(jnp.int32)
    emb_flat, wo_flat = _sc_embed(idx, token_embeddings, position_embeddings)
    emb = emb_flat.reshape(BATCH, SEQ, D_MODEL)
    wo = wo_flat.reshape(BATCH, SEQ, D_MODEL)
    return (emb, None, wo)
